# Initial kernel scaffold; baseline (speedup 1.0000x reference)
import jax
import jax.numpy as jnp
from jax.experimental import pallas as pl


def kernel(x, edge_index, edge_attr, batch, node_emb, edge_emb, enc_W, enc_b, pre_W, pre_b, post_W, post_b, lin_W, lin_b, bn_g, bn_b, mlp_W1, mlp_b1, mlp_W2, mlp_b2, mlp_W3, mlp_b3):
    raise NotImplementedError("write your pallas kernel here")


# speedup vs baseline: 14.8203x; 14.8203x over previous
import functools
import numpy as np
import jax
import jax.numpy as jnp
from jax import lax
from jax.experimental import pallas as pl
from jax.experimental.pallas import tpu as pltpu

N = 10000
E = 160000
G = 128
L = 4
T = 5
F = 75
FOUT = 15

_DEG = np.array([0, 120, 340, 780, 1450, 2300, 3200, 4100, 4800, 5200, 5300,
                 5100, 4600, 3900, 3100, 2300, 1600, 1050, 640, 360, 190, 95,
                 45, 20, 8, 2], dtype=np.float64)
AVG_LOG = float((np.log(np.arange(len(_DEG)) + 1.0) * _DEG).sum() / _DEG.sum())

NPAD = 10240
WN = 320
EPAD = E + 64
W384 = 384
ROW = 4 * W384
NB = 256
NGRID = NPAD // NB
SQRT1EM5 = float(np.sqrt(1e-5))


def _pad2(a, r, c):
    return jnp.pad(a, ((0, r - a.shape[0]), (0, c - a.shape[1])))


def _pad1(a, n):
    return jnp.pad(a, (0, n - a.shape[0]))


def _embed_body(x_ref, emb_ref, o_ref):
    xb = x_ref[...]
    cols = lax.broadcasted_iota(jnp.int32, (NB, 128), 1)
    oh = (xb == cols).astype(jnp.float32)
    o_ref[...] = jnp.dot(oh, emb_ref[...], preferred_element_type=jnp.float32)


def _embed(x_pad, emb_pad):
    return pl.pallas_call(
        _embed_body,
        grid=(NGRID,),
        in_specs=[pl.BlockSpec((NB, 128), lambda i: (i, 0)),
                  pl.BlockSpec((128, 128), lambda i: (0, 0))],
        out_specs=pl.BlockSpec((NB, 128), lambda i: (i, 0)),
        out_shape=jax.ShapeDtypeStruct((NPAD, 128), jnp.float32),
    )(x_pad, emb_pad)


def _mm_body(h_ref, w_ref, o_ref):
    o_ref[...] = jnp.dot(h_ref[...], w_ref[...],
                         preferred_element_type=jnp.float32)


def _node_mm(h, w):
    return pl.pallas_call(
        _mm_body,
        grid=(NGRID,),
        in_specs=[pl.BlockSpec((NB, 128), lambda i: (i, 0)),
                  pl.BlockSpec((128, W384), lambda i: (0, 0))],
        out_specs=pl.BlockSpec((NB, W384), lambda i: (i, 0)),
        out_shape=jax.ShapeDtypeStruct((NPAD, W384), jnp.float32),
    )(h, w)


def _post_body(h_ref, st_ref, cnt_ref, wd_ref, wx_ref, wa_ref, wb_ref,
               wc_ref, bp_ref, lw_ref, lb_ref, z_ref, bn_ref, acc_ref):
    i = pl.program_id(0)

    @pl.when(i == 0)
    def _():
        acc_ref[...] = jnp.zeros_like(acc_ref)

    cnt = cnt_ref[:, 0:1]
    deg = jnp.maximum(cnt, 1.0)
    logd = jnp.log(deg + 1.0)
    amp = logd / AVG_LOG
    att = AVG_LOG / logd
    has = cnt > 0.0

    h = h_ref[...]
    hd = jnp.dot(h, wd_ref[...], preferred_element_type=jnp.float32)
    d1 = deg
    S = st_ref[:, 0:W384]
    Q = st_ref[:, W384:2 * W384]
    mnv = st_ref[:, 2 * W384:3 * W384]
    mxv = st_ref[:, 3 * W384:4 * W384]
    sm = S / d1
    mean = jnp.where(has, hd + sm, 0.0)
    std = jnp.where(has, jnp.sqrt(jax.nn.relu(Q / d1 - sm * sm) + 1e-5),
                    SQRT1EM5)
    mn = jnp.where(has, hd + mnv, 0.0)
    mx = jnp.where(has, hd + mxv, 0.0)
    agg = jnp.concatenate([mean, mn, mx, std], axis=1)
    pa = jnp.dot(agg, wa_ref[...], preferred_element_type=jnp.float32)
    pb = jnp.dot(agg, wb_ref[...], preferred_element_type=jnp.float32)
    pc = jnp.dot(agg, wc_ref[...], preferred_element_type=jnp.float32)
    z75 = (jnp.dot(h, wx_ref[...], preferred_element_type=jnp.float32)
           + pa + amp * pb + att * pc + bp_ref[...][None, :])
    z = jnp.dot(z75, lw_ref[...], preferred_element_type=jnp.float32) \
        + lb_ref[...][None, :]
    z_ref[...] = z

    rows = i * NB + lax.broadcasted_iota(jnp.int32, (NB, 1), 0)
    zm = jnp.where(rows < N, z, 0.0)
    acc_ref[0:1, :] += jnp.sum(zm, axis=0, keepdims=True)
    acc_ref[1:2, :] += jnp.sum(zm * zm, axis=0, keepdims=True)
    bn_ref[...] = acc_ref[...]


def _post(h, stats, cnt, wd, wx, wa, wb, wc, bp, lw, lb):
    return pl.pallas_call(
        _post_body,
        grid=(NGRID,),
        in_specs=[pl.BlockSpec((NB, 128), lambda i: (i, 0)),
                  pl.BlockSpec((NB, ROW), lambda i: (i, 0)),
                  pl.BlockSpec((NB, 128), lambda i: (i, 0)),
                  pl.BlockSpec((128, W384), lambda i: (0, 0)),
                  pl.BlockSpec((128, 128), lambda i: (0, 0)),
                  pl.BlockSpec((ROW, 128), lambda i: (0, 0)),
                  pl.BlockSpec((ROW, 128), lambda i: (0, 0)),
                  pl.BlockSpec((ROW, 128), lambda i: (0, 0)),
                  pl.BlockSpec((128,), lambda i: (0,)),
                  pl.BlockSpec((128, 128), lambda i: (0, 0)),
                  pl.BlockSpec((128,), lambda i: (0,))],
        out_specs=[pl.BlockSpec((NB, 128), lambda i: (i, 0)),
                   pl.BlockSpec((2, 128), lambda i: (0, 0))],
        out_shape=[jax.ShapeDtypeStruct((NPAD, 128), jnp.float32),
                   jax.ShapeDtypeStruct((2, 128), jnp.float32)],
        scratch_shapes=[pltpu.VMEM((2, 128), jnp.float32)],
    )(h, stats, cnt, wd, wx, wa, wb, wc, bp, lw, lb)


def _bn_body(z_ref, s_ref, g_ref, b_ref, o_ref):
    s = s_ref[...]
    mu = s[0:1, :] / float(N)
    var = s[1:2, :] / float(N) - mu * mu
    scale = g_ref[...][None, :] * lax.rsqrt(var + 1e-5)
    o_ref[...] = jax.nn.relu((z_ref[...] - mu) * scale + b_ref[...][None, :])


def _bn_relu(z, sums, g, b):
    return pl.pallas_call(
        _bn_body,
        grid=(NGRID,),
        in_specs=[pl.BlockSpec((NB, 128), lambda i: (i, 0)),
                  pl.BlockSpec((2, 128), lambda i: (0, 0)),
                  pl.BlockSpec((128,), lambda i: (0,)),
                  pl.BlockSpec((128,), lambda i: (0,))],
        out_specs=pl.BlockSpec((NB, 128), lambda i: (i, 0)),
        out_shape=jax.ShapeDtypeStruct((NPAD, 128), jnp.float32),
    )(z, sums, g, b)


def _pool_body(h_ref, b_ref, w1_ref, b1_ref, w2_ref, b2_ref, w3_ref, b3_ref,
               o_ref, acc_ref):
    i = pl.program_id(0)

    @pl.when(i == 0)
    def _():
        acc_ref[...] = jnp.zeros_like(acc_ref)

    bb = b_ref[...]
    rows = i * NB + lax.broadcasted_iota(jnp.int32, (NB, 128), 0)
    cols = lax.broadcasted_iota(jnp.int32, (NB, 128), 1)
    oh = ((bb == cols) & (rows < N)).astype(jnp.float32)
    acc_ref[...] += lax.dot_general(oh, h_ref[...],
                                    (((0,), (0,)), ((), ())),
                                    preferred_element_type=jnp.float32)

    @pl.when(i == NGRID - 1)
    def _():
        g0 = acc_ref[...]
        g1 = jax.nn.relu(jnp.dot(g0, w1_ref[...],
                                 preferred_element_type=jnp.float32)
                         + b1_ref[...][None, :])
        g2 = jax.nn.relu(jnp.dot(g1, w2_ref[...],
                                 preferred_element_type=jnp.float32)
                         + b2_ref[...][None, :])
        g3 = jnp.dot(g2, w3_ref[...], preferred_element_type=jnp.float32) \
            + b3_ref[...][None, :]
        o_ref[...] = g3[:, 0:1]


def _pool_mlp(h, batch_pad, w1, b1, w2, b2, w3, b3):
    return pl.pallas_call(
        _pool_body,
        grid=(NGRID,),
        in_specs=[pl.BlockSpec((NB, 128), lambda i: (i, 0)),
                  pl.BlockSpec((NB, 128), lambda i: (i, 0)),
                  pl.BlockSpec((128, 128), lambda i: (0, 0)),
                  pl.BlockSpec((128,), lambda i: (0,)),
                  pl.BlockSpec((128, 128), lambda i: (0, 0)),
                  pl.BlockSpec((128,), lambda i: (0,)),
                  pl.BlockSpec((128, 128), lambda i: (0, 0)),
                  pl.BlockSpec((128,), lambda i: (0,))],
        out_specs=pl.BlockSpec((G, 1), lambda i: (0, 0)),
        out_shape=jax.ShapeDtypeStruct((G, 1), jnp.float32),
        scratch_shapes=[pltpu.VMEM((128, 128), jnp.float32)],
    )(h, batch_pad, w1, b1, w2, b2, w3, b3)


def _edge_stats(hs, src_s, attr_s, dst_s, ctab):
    v = hs[src_s] + ctab[attr_s]
    S = jax.ops.segment_sum(v, dst_s, num_segments=N)
    Q = jax.ops.segment_sum(v * v, dst_s, num_segments=N)
    mnv = jax.ops.segment_min(v, dst_s, num_segments=N)
    mxv = jax.ops.segment_max(v, dst_s, num_segments=N)
    st = jnp.concatenate([S, Q, mnv, mxv], axis=1)
    return jnp.pad(st, ((0, NPAD - N), (0, 0)))


def kernel(x, edge_index, edge_attr, batch, node_emb, edge_emb, enc_W, enc_b,
           pre_W, pre_b, post_W, post_b, lin_W, lin_b, bn_g, bn_b,
           mlp_W1, mlp_b1, mlp_W2, mlp_b2, mlp_W3, mlp_b3):
    src = edge_index[0].astype(jnp.int32)
    dst = edge_index[1].astype(jnp.int32)
    order = jnp.argsort(dst)
    dst_s = dst[order]
    src_s = src[order]
    attr_s = edge_attr[order].astype(jnp.int32)
    offsets = jnp.searchsorted(dst_s, jnp.arange(N + 1)).astype(jnp.int32)
    counts = (offsets[1:] - offsets[:-1]).astype(jnp.float32)
    cnt_pad = jnp.broadcast_to(_pad1(counts, NPAD)[:, None], (NPAD, 128))

    x_pad = jnp.broadcast_to(
        _pad1(jnp.squeeze(x, axis=1).astype(jnp.int32), NPAD)[:, None],
        (NPAD, 128))
    batch_pad = jnp.broadcast_to(
        _pad1(batch.astype(jnp.int32), NPAD)[:, None], (NPAD, 128))
    emb_pad = _pad2(node_emb, 128, 128)

    h = _embed(x_pad, emb_pad)

    for l in range(L):
        e4 = edge_emb @ enc_W[l] + enc_b[l]
        ctab = (jnp.einsum('af,tfg->atg', e4, pre_W[l][:, 2 * F:3 * F, :])
                + pre_b[l]).reshape(4, T * F)
        ctab_pad = _pad2(ctab, 4, W384)
        ws = _pad2(pre_W[l][:, F:2 * F, :].transpose(1, 0, 2).reshape(F, T * F),
                   128, W384)
        wd = _pad2(pre_W[l][:, 0:F, :].transpose(1, 0, 2).reshape(F, T * F),
                   128, W384)
        wx = _pad2(post_W[l][:, 0:F, :].transpose(1, 0, 2).reshape(F, T * FOUT),
                   128, 128)

        def _blockdiag(wsc):
            tmp = wsc.reshape(T, 4, F, FOUT).transpose(1, 0, 2, 3)
            bd = tmp[:, :, :, None, :] * jnp.eye(T)[None, :, None, :, None]
            bd = bd.reshape(4, T * F, T * FOUT)
            return jnp.concatenate([_pad2(bd[s], W384, 128) for s in range(4)],
                                   axis=0)

        wa = _blockdiag(post_W[l][:, F + 0 * 4 * F + 0:F + 1 * 4 * F, :])
        wb = _blockdiag(post_W[l][:, F + 1 * 4 * F:F + 2 * 4 * F, :])
        wc = _blockdiag(post_W[l][:, F + 2 * 4 * F:F + 3 * 4 * F, :])
        bp = _pad1(post_b[l].reshape(T * FOUT), 128)
        lw = _pad2(lin_W[l], 128, 128)
        lb = _pad1(lin_b[l], 128)
        bg = _pad1(bn_g[l], 128)
        bb = _pad1(bn_b[l], 128)

        hs = _node_mm(h, ws)
        stats = _edge_stats(hs[:N], src_s, attr_s, dst_s, ctab_pad)
        z, sums = _post(h, stats, cnt_pad, wd, wx, wa, wb, wc, bp, lw, lb)
        h = _bn_relu(z, sums, bg, bb)

    w1 = _pad2(mlp_W1, 128, 128)
    b1 = _pad1(mlp_b1, 128)
    w2 = _pad2(mlp_W2, 128, 128)
    b2 = _pad1(mlp_b2, 128)
    w3 = _pad2(mlp_W3, 128, 128)
    b3 = _pad1(mlp_b3, 128)
    return _pool_mlp(h, batch_pad, w1, b1, w2, b2, w3, b3)


# speedup vs baseline: 27.3045x; 1.8424x over previous
import functools
import numpy as np
import jax
import jax.numpy as jnp
from jax import lax
from jax.experimental import pallas as pl
from jax.experimental.pallas import tpu as pltpu
from jax.experimental.pallas import tpu_sc as plsc

N = 10000
E = 160000
G = 128
L = 4
T = 5
F = 75
FOUT = 15

_DEG = np.array([0, 120, 340, 780, 1450, 2300, 3200, 4100, 4800, 5200, 5300,
                 5100, 4600, 3900, 3100, 2300, 1600, 1050, 640, 360, 190, 95,
                 45, 20, 8, 2], dtype=np.float64)
AVG_LOG = float((np.log(np.arange(len(_DEG)) + 1.0) * _DEG).sum() / _DEG.sum())

NPAD = 10240
WN = 320
EPAD = E + 64
W384 = 384
ROW = 4 * W384
NB = 256
NGRID = NPAD // NB
SQRT1EM5 = float(np.sqrt(1e-5))


def _pad2(a, r, c):
    return jnp.pad(a, ((0, r - a.shape[0]), (0, c - a.shape[1])))


def _pad1(a, n):
    return jnp.pad(a, (0, n - a.shape[0]))


def _embed_body(x_ref, emb_ref, o_ref):
    xb = x_ref[...]
    cols = lax.broadcasted_iota(jnp.int32, (NB, 128), 1)
    oh = (xb == cols).astype(jnp.float32)
    o_ref[...] = jnp.dot(oh, emb_ref[...], preferred_element_type=jnp.float32)


def _embed(x_pad, emb_pad):
    return pl.pallas_call(
        _embed_body,
        grid=(NGRID,),
        in_specs=[pl.BlockSpec((NB, 128), lambda i: (i, 0)),
                  pl.BlockSpec((128, 128), lambda i: (0, 0))],
        out_specs=pl.BlockSpec((NB, 128), lambda i: (i, 0)),
        out_shape=jax.ShapeDtypeStruct((NPAD, 128), jnp.float32),
    )(x_pad, emb_pad)


def _mm_body(h_ref, w_ref, o_ref):
    o_ref[...] = jnp.dot(h_ref[...], w_ref[...],
                         preferred_element_type=jnp.float32)


def _node_mm(h, w):
    return pl.pallas_call(
        _mm_body,
        grid=(NGRID,),
        in_specs=[pl.BlockSpec((NB, 128), lambda i: (i, 0)),
                  pl.BlockSpec((128, W384), lambda i: (0, 0))],
        out_specs=pl.BlockSpec((NB, W384), lambda i: (i, 0)),
        out_shape=jax.ShapeDtypeStruct((NPAD, W384), jnp.float32),
    )(h, w)


def _post_body(h_ref, st_ref, cnt_ref, wd_ref, wx_ref, wa_ref, wb_ref,
               wc_ref, bp_ref, lw_ref, lb_ref, z_ref, bn_ref, acc_ref):
    i = pl.program_id(0)

    @pl.when(i == 0)
    def _():
        acc_ref[...] = jnp.zeros_like(acc_ref)

    cnt = cnt_ref[:, 0:1]
    deg = jnp.maximum(cnt, 1.0)
    logd = jnp.log(deg + 1.0)
    amp = logd / AVG_LOG
    att = AVG_LOG / logd
    has = cnt > 0.0

    h = h_ref[...]
    hd = jnp.dot(h, wd_ref[...], preferred_element_type=jnp.float32)
    d1 = deg
    S = st_ref[:, 0:W384]
    Q = st_ref[:, W384:2 * W384]
    mnv = st_ref[:, 2 * W384:3 * W384]
    mxv = st_ref[:, 3 * W384:4 * W384]
    sm = S / d1
    mean = jnp.where(has, hd + sm, 0.0)
    std = jnp.where(has, jnp.sqrt(jax.nn.relu(Q / d1 - sm * sm) + 1e-5),
                    SQRT1EM5)
    mn = jnp.where(has, hd + mnv, 0.0)
    mx = jnp.where(has, hd + mxv, 0.0)
    agg = jnp.concatenate([mean, mn, mx, std], axis=1)
    pa = jnp.dot(agg, wa_ref[...], preferred_element_type=jnp.float32)
    pb = jnp.dot(agg, wb_ref[...], preferred_element_type=jnp.float32)
    pc = jnp.dot(agg, wc_ref[...], preferred_element_type=jnp.float32)
    z75 = (jnp.dot(h, wx_ref[...], preferred_element_type=jnp.float32)
           + pa + amp * pb + att * pc + bp_ref[...][None, :])
    z = jnp.dot(z75, lw_ref[...], preferred_element_type=jnp.float32) \
        + lb_ref[...][None, :]
    z_ref[...] = z

    rows = i * NB + lax.broadcasted_iota(jnp.int32, (NB, 1), 0)
    zm = jnp.where(rows < N, z, 0.0)
    acc_ref[0:1, :] += jnp.sum(zm, axis=0, keepdims=True)
    acc_ref[1:2, :] += jnp.sum(zm * zm, axis=0, keepdims=True)
    bn_ref[...] = acc_ref[...]


def _post(h, stats, cnt, wd, wx, wa, wb, wc, bp, lw, lb):
    return pl.pallas_call(
        _post_body,
        grid=(NGRID,),
        in_specs=[pl.BlockSpec((NB, 128), lambda i: (i, 0)),
                  pl.BlockSpec((NB, ROW), lambda i: (i, 0)),
                  pl.BlockSpec((NB, 128), lambda i: (i, 0)),
                  pl.BlockSpec((128, W384), lambda i: (0, 0)),
                  pl.BlockSpec((128, 128), lambda i: (0, 0)),
                  pl.BlockSpec((ROW, 128), lambda i: (0, 0)),
                  pl.BlockSpec((ROW, 128), lambda i: (0, 0)),
                  pl.BlockSpec((ROW, 128), lambda i: (0, 0)),
                  pl.BlockSpec((128,), lambda i: (0,)),
                  pl.BlockSpec((128, 128), lambda i: (0, 0)),
                  pl.BlockSpec((128,), lambda i: (0,))],
        out_specs=[pl.BlockSpec((NB, 128), lambda i: (i, 0)),
                   pl.BlockSpec((2, 128), lambda i: (0, 0))],
        out_shape=[jax.ShapeDtypeStruct((NPAD, 128), jnp.float32),
                   jax.ShapeDtypeStruct((2, 128), jnp.float32)],
        scratch_shapes=[pltpu.VMEM((2, 128), jnp.float32)],
    )(h, stats, cnt, wd, wx, wa, wb, wc, bp, lw, lb)


def _bn_body(z_ref, s_ref, g_ref, b_ref, o_ref):
    s = s_ref[...]
    mu = s[0:1, :] / float(N)
    var = s[1:2, :] / float(N) - mu * mu
    scale = g_ref[...][None, :] * lax.rsqrt(var + 1e-5)
    o_ref[...] = jax.nn.relu((z_ref[...] - mu) * scale + b_ref[...][None, :])


def _bn_relu(z, sums, g, b):
    return pl.pallas_call(
        _bn_body,
        grid=(NGRID,),
        in_specs=[pl.BlockSpec((NB, 128), lambda i: (i, 0)),
                  pl.BlockSpec((2, 128), lambda i: (0, 0)),
                  pl.BlockSpec((128,), lambda i: (0,)),
                  pl.BlockSpec((128,), lambda i: (0,))],
        out_specs=pl.BlockSpec((NB, 128), lambda i: (i, 0)),
        out_shape=jax.ShapeDtypeStruct((NPAD, 128), jnp.float32),
    )(z, sums, g, b)


def _pool_body(h_ref, b_ref, w1_ref, b1_ref, w2_ref, b2_ref, w3_ref, b3_ref,
               o_ref, acc_ref):
    i = pl.program_id(0)

    @pl.when(i == 0)
    def _():
        acc_ref[...] = jnp.zeros_like(acc_ref)

    bb = b_ref[...]
    rows = i * NB + lax.broadcasted_iota(jnp.int32, (NB, 128), 0)
    cols = lax.broadcasted_iota(jnp.int32, (NB, 128), 1)
    oh = ((bb == cols) & (rows < N)).astype(jnp.float32)
    acc_ref[...] += lax.dot_general(oh, h_ref[...],
                                    (((0,), (0,)), ((), ())),
                                    preferred_element_type=jnp.float32)

    @pl.when(i == NGRID - 1)
    def _():
        g0 = acc_ref[...]
        g1 = jax.nn.relu(jnp.dot(g0, w1_ref[...],
                                 preferred_element_type=jnp.float32)
                         + b1_ref[...][None, :])
        g2 = jax.nn.relu(jnp.dot(g1, w2_ref[...],
                                 preferred_element_type=jnp.float32)
                         + b2_ref[...][None, :])
        g3 = jnp.dot(g2, w3_ref[...], preferred_element_type=jnp.float32) \
            + b3_ref[...][None, :]
        o_ref[...] = g3[:, 0:1]


def _pool_mlp(h, batch_pad, w1, b1, w2, b2, w3, b3):
    return pl.pallas_call(
        _pool_body,
        grid=(NGRID,),
        in_specs=[pl.BlockSpec((NB, 128), lambda i: (i, 0)),
                  pl.BlockSpec((NB, 128), lambda i: (i, 0)),
                  pl.BlockSpec((128, 128), lambda i: (0, 0)),
                  pl.BlockSpec((128,), lambda i: (0,)),
                  pl.BlockSpec((128, 128), lambda i: (0, 0)),
                  pl.BlockSpec((128,), lambda i: (0,)),
                  pl.BlockSpec((128, 128), lambda i: (0, 0)),
                  pl.BlockSpec((128,), lambda i: (0,))],
        out_specs=pl.BlockSpec((G, 1), lambda i: (0, 0)),
        out_shape=jax.ShapeDtypeStruct((G, 1), jnp.float32),
        scratch_shapes=[pltpu.VMEM((128, 128), jnp.float32)],
    )(h, batch_pad, w1, b1, w2, b2, w3, b3)


CH = 32
SB = 16
SLOTS = 48
HALF = 192
NGRP = HALF // 16
FMAX = float(np.float32(3.0e38))


def _sc_walk(h, cur_in, d_e, a_e, i, accs, d0, d1, rows_v, c_v, stage_v,
             out_hbm, do_stream):
    def flush(args):
        cur = args[0]
        acc = args[1:]
        valid = jnp.logical_and(cur >= d0, cur < d1)

        @pl.when(valid)
        def _():
            slot = lax.rem(cur, SLOTS)
            for st in range(4):
                for k in range(NGRP):
                    off = slot * ROW + st * W384 + h * HALF + k * 16
                    stage_v[pl.ds(off, 16)] = acc[st * NGRP + k]
            if do_stream:
                cross = (d_e >> 4) != (cur >> 4)

                @pl.when(cross)
                def _():
                    blk = cur >> 4
                    sbase = pl.multiple_of(lax.rem(blk, 3) * (SB * ROW), 8)
                    obase = pl.multiple_of(blk * (SB * ROW), 8)
                    pltpu.sync_copy(stage_v.at[pl.ds(sbase, SB * ROW)],
                                    out_hbm.at[pl.ds(obase, SB * ROW)])

        zero = jnp.zeros((16,), jnp.float32)
        init = ([zero] * (2 * NGRP)
                + [jnp.full((16,), FMAX, jnp.float32)] * NGRP
                + [jnp.full((16,), -FMAX, jnp.float32)] * NGRP)
        return (d_e, *init)

    carry = lax.cond(d_e != cur_in, flush, lambda a: a, (cur_in, *accs))
    cur = carry[0]
    acc = list(carry[1:])
    for k in range(NGRP):
        off = h * HALF + k * 16
        row = rows_v[i, pl.ds(off, 16)]
        c = c_v[pl.ds(a_e * W384 + off, 16)]
        v = row + c
        acc[k] = acc[k] + v
        acc[NGRP + k] = acc[NGRP + k] + v * v
        acc[2 * NGRP + k] = jnp.minimum(acc[2 * NGRP + k], v)
        acc[3 * NGRP + k] = jnp.maximum(acc[3 * NGRP + k], v)
    return (cur, *acc)


def _edge_agg(hs, src_s, dst_s, attr_s, wstart, ctab_flat):
    mesh = plsc.VectorSubcoreMesh(core_axis_name="c", subcore_axis_name="s")

    @functools.partial(
        pl.kernel, mesh=mesh,
        out_type=jax.ShapeDtypeStruct((NPAD * ROW,), jnp.float32),
        scratch_types=[
            pltpu.VMEM((CH,), jnp.int32),
            pltpu.VMEM((CH + 16,), jnp.int32),
            pltpu.VMEM((CH + 16,), jnp.int32),
            pltpu.VMEM((CH, W384), jnp.float32),
            pltpu.VMEM((SLOTS * ROW,), jnp.float32),
            pltpu.VMEM((4 * W384,), jnp.float32),
            pltpu.VMEM((272,), jnp.int32),
            pltpu.SemaphoreType.DMA,
        ])
    def k(hs_hbm, src_hbm, dst_hbm, attr_hbm, ws_hbm, c_hbm, out_hbm,
          idx_v, dst_v, attr_v, rows_v, stage_v, c_v, ws_v, sem):
        wid = lax.axis_index("s") * 2 + lax.axis_index("c")
        d0 = wid * WN
        d1 = d0 + WN
        pltpu.sync_copy(ws_hbm, ws_v.at[pl.ds(0, 256)])
        pltpu.sync_copy(c_hbm, c_v)
        wwin = ws_v[pl.ds(wid * 8, 16)]
        e0 = wwin[0]
        e1 = wwin[1]
        e0a = pl.multiple_of((e0 >> 3) << 3, 8)
        nch = (e1 - e0a + CH - 1) // CH

        zero = jnp.zeros((16,), jnp.float32)
        init = ([zero] * (2 * NGRP)
                + [jnp.full((16,), FMAX, jnp.float32)] * NGRP
                + [jnp.full((16,), -FMAX, jnp.float32)] * NGRP)

        def chunk_body(ch, carry):
            cur0 = carry[0]
            cur1 = carry[1]
            acc0 = list(carry[2:2 + 4 * NGRP])
            acc1 = list(carry[2 + 4 * NGRP:])
            base = pl.multiple_of(e0a + ch * CH, 8)
            pltpu.sync_copy(src_hbm.at[pl.ds(base, CH)], idx_v)
            pltpu.sync_copy(dst_hbm.at[pl.ds(base, CH)], dst_v.at[pl.ds(0, CH)])
            pltpu.sync_copy(attr_hbm.at[pl.ds(base, CH)],
                            attr_v.at[pl.ds(0, CH)])
            pltpu.async_copy(hs_hbm.at[idx_v], rows_v, sem).wait()

            def mk_body(h, do_stream):
                def body(g, c):
                    dwin = dst_v[pl.ds(g * 8, 16)]
                    awin = attr_v[pl.ds(g * 8, 16)]
                    for j in range(8):
                        c = _sc_walk(h, c[0], dwin[j], awin[j], g * 8 + j,
                                     c[1:], d0, d1, rows_v, c_v, stage_v,
                                     out_hbm, do_stream)
                    return c
                return body

            r0 = lax.fori_loop(0, CH // 8, mk_body(0, False), (cur0, *acc0))
            r1 = lax.fori_loop(0, CH // 8, mk_body(1, True), (cur1, *acc1))
            return (r0[0], r1[0], *r0[1:], *r1[1:])

        fin = lax.fori_loop(0, nch, chunk_body,
                            (jnp.int32(-1), jnp.int32(-1), *init, *init))
        cur0 = fin[0]
        cur1 = fin[1]
        acc0 = fin[2:2 + 4 * NGRP]
        acc1 = fin[2 + 4 * NGRP:]

        valid = jnp.logical_and(cur1 >= d0, cur1 < d1)

        @pl.when(valid)
        def _():
            slot = lax.rem(cur1, SLOTS)
            for h, acc in ((0, acc0), (1, acc1)):
                for st in range(4):
                    for k2 in range(NGRP):
                        off = slot * ROW + st * W384 + h * HALF + k2 * 16
                        stage_v[pl.ds(off, 16)] = acc[st * NGRP + k2]
            blk = cur1 >> 4
            sbase = pl.multiple_of(lax.rem(blk, 3) * (SB * ROW), 8)
            obase = pl.multiple_of(blk * (SB * ROW), 8)
            pltpu.sync_copy(stage_v.at[pl.ds(sbase, SB * ROW)],
                            out_hbm.at[pl.ds(obase, SB * ROW)])

    return k(hs, src_s, dst_s, attr_s, wstart, ctab_flat)


def kernel(x, edge_index, edge_attr, batch, node_emb, edge_emb, enc_W, enc_b,
           pre_W, pre_b, post_W, post_b, lin_W, lin_b, bn_g, bn_b,
           mlp_W1, mlp_b1, mlp_W2, mlp_b2, mlp_W3, mlp_b3):
    src = edge_index[0].astype(jnp.int32)
    dst = edge_index[1].astype(jnp.int32)
    order = jnp.argsort(dst)
    dst_s = dst[order]
    src_s = src[order]
    attr_s = edge_attr[order].astype(jnp.int32)
    offsets = jnp.searchsorted(dst_s, jnp.arange(N + 1)).astype(jnp.int32)
    counts = (offsets[1:] - offsets[:-1]).astype(jnp.float32)
    src_sp = _pad1(src_s, EPAD)
    dst_sp = jnp.pad(dst_s, (0, EPAD - E), constant_values=NPAD)
    attr_sp = _pad1(attr_s, EPAD)
    wpairs = jnp.zeros((32, 8), jnp.int32)
    wpairs = wpairs.at[:, 0].set(offsets[jnp.arange(32) * WN])
    wpairs = wpairs.at[:, 1].set(
        offsets[jnp.minimum((jnp.arange(32) + 1) * WN, N)])
    wpairs = wpairs.reshape(-1)
    cnt_pad = jnp.broadcast_to(_pad1(counts, NPAD)[:, None], (NPAD, 128))

    x_pad = jnp.broadcast_to(
        _pad1(jnp.squeeze(x, axis=1).astype(jnp.int32), NPAD)[:, None],
        (NPAD, 128))
    batch_pad = jnp.broadcast_to(
        _pad1(batch.astype(jnp.int32), NPAD)[:, None], (NPAD, 128))
    emb_pad = _pad2(node_emb, 128, 128)

    h = _embed(x_pad, emb_pad)

    for l in range(L):
        e4 = edge_emb @ enc_W[l] + enc_b[l]
        ctab = (jnp.einsum('af,tfg->atg', e4, pre_W[l][:, 2 * F:3 * F, :])
                + pre_b[l]).reshape(4, T * F)
        ctab_pad = _pad2(ctab, 4, W384)
        ws = _pad2(pre_W[l][:, F:2 * F, :].transpose(1, 0, 2).reshape(F, T * F),
                   128, W384)
        wd = _pad2(pre_W[l][:, 0:F, :].transpose(1, 0, 2).reshape(F, T * F),
                   128, W384)
        wx = _pad2(post_W[l][:, 0:F, :].transpose(1, 0, 2).reshape(F, T * FOUT),
                   128, 128)

        def _blockdiag(wsc):
            tmp = wsc.reshape(T, 4, F, FOUT).transpose(1, 0, 2, 3)
            bd = tmp[:, :, :, None, :] * jnp.eye(T)[None, :, None, :, None]
            bd = bd.reshape(4, T * F, T * FOUT)
            return jnp.concatenate([_pad2(bd[s], W384, 128) for s in range(4)],
                                   axis=0)

        wa = _blockdiag(post_W[l][:, F + 0 * 4 * F + 0:F + 1 * 4 * F, :])
        wb = _blockdiag(post_W[l][:, F + 1 * 4 * F:F + 2 * 4 * F, :])
        wc = _blockdiag(post_W[l][:, F + 2 * 4 * F:F + 3 * 4 * F, :])
        bp = _pad1(post_b[l].reshape(T * FOUT), 128)
        lw = _pad2(lin_W[l], 128, 128)
        lb = _pad1(lin_b[l], 128)
        bg = _pad1(bn_g[l], 128)
        bb = _pad1(bn_b[l], 128)

        hs = _node_mm(h, ws)
        stats = _edge_agg(hs, src_sp, dst_sp, attr_sp, wpairs,
                          ctab_pad.reshape(-1)).reshape(NPAD, ROW)
        z, sums = _post(h, stats, cnt_pad, wd, wx, wa, wb, wc, bp, lw, lb)
        h = _bn_relu(z, sums, bg, bb)

    w1 = _pad2(mlp_W1, 128, 128)
    b1 = _pad1(mlp_b1, 128)
    w2 = _pad2(mlp_W2, 128, 128)
    b2 = _pad1(mlp_b2, 128)
    w3 = _pad2(mlp_W3, 128, 128)
    b3 = _pad1(mlp_b3, 128)
    return _pool_mlp(h, batch_pad, w1, b1, w2, b2, w3, b3)


# speedup vs baseline: 46.6701x; 1.7092x over previous
import functools
import numpy as np
import jax
import jax.numpy as jnp
from jax import lax
from jax.experimental import pallas as pl
from jax.experimental.pallas import tpu as pltpu
from jax.experimental.pallas import tpu_sc as plsc

N = 10000
E = 160000
G = 128
L = 4
T = 5
F = 75
FOUT = 15

_DEG = np.array([0, 120, 340, 780, 1450, 2300, 3200, 4100, 4800, 5200, 5300,
                 5100, 4600, 3900, 3100, 2300, 1600, 1050, 640, 360, 190, 95,
                 45, 20, 8, 2], dtype=np.float64)
AVG_LOG = float((np.log(np.arange(len(_DEG)) + 1.0) * _DEG).sum() / _DEG.sum())

NPAD = 10240
WN = 320
EPAD = E + 64
W384 = 384
ROW = 4 * W384
NB = 256
NGRID = NPAD // NB
SQRT1EM5 = float(np.sqrt(1e-5))


def _pad2(a, r, c):
    return jnp.pad(a, ((0, r - a.shape[0]), (0, c - a.shape[1])))


def _pad1(a, n):
    return jnp.pad(a, (0, n - a.shape[0]))


def _embed_body(x_ref, emb_ref, o_ref):
    xb = x_ref[...]
    cols = lax.broadcasted_iota(jnp.int32, (NB, 128), 1)
    oh = (xb == cols).astype(jnp.float32)
    o_ref[...] = jnp.dot(oh, emb_ref[...], preferred_element_type=jnp.float32, precision=lax.Precision.HIGHEST)


def _embed(x_pad, emb_pad):
    return pl.pallas_call(
        _embed_body,
        grid=(NGRID,),
        in_specs=[pl.BlockSpec((NB, 128), lambda i: (i, 0)),
                  pl.BlockSpec((128, 128), lambda i: (0, 0))],
        out_specs=pl.BlockSpec((NB, 128), lambda i: (i, 0)),
        out_shape=jax.ShapeDtypeStruct((NPAD, 128), jnp.float32),
    )(x_pad, emb_pad)


def _mm_body(h_ref, w_ref, o_ref):
    o_ref[...] = jnp.dot(h_ref[...], w_ref[...],
                         preferred_element_type=jnp.float32, precision=lax.Precision.HIGHEST)


def _node_mm(h, w):
    return pl.pallas_call(
        _mm_body,
        grid=(NGRID,),
        in_specs=[pl.BlockSpec((NB, 128), lambda i: (i, 0)),
                  pl.BlockSpec((128, W384), lambda i: (0, 0))],
        out_specs=pl.BlockSpec((NB, W384), lambda i: (i, 0)),
        out_shape=jax.ShapeDtypeStruct((NPAD, W384), jnp.float32),
    )(h, w)


def _post_body(h_ref, st_ref, cnt_ref, wd_ref, wx_ref, wa_ref, wb_ref,
               wc_ref, bp_ref, lw_ref, lb_ref, z_ref, bn_ref, acc_ref):
    i = pl.program_id(0)

    @pl.when(i == 0)
    def _():
        acc_ref[...] = jnp.zeros_like(acc_ref)

    cnt = cnt_ref[:, 0:1]
    deg = jnp.maximum(cnt, 1.0)
    logd = jnp.log(deg + 1.0)
    amp = logd / AVG_LOG
    att = AVG_LOG / logd
    has = cnt > 0.0

    h = h_ref[...]
    hd = jnp.dot(h, wd_ref[...], preferred_element_type=jnp.float32, precision=lax.Precision.HIGHEST)
    d1 = deg
    S = st_ref[:, 0:W384]
    Q = st_ref[:, W384:2 * W384]
    mnv = st_ref[:, 2 * W384:3 * W384]
    mxv = st_ref[:, 3 * W384:4 * W384]
    sm = S / d1
    mean = jnp.where(has, hd + sm, 0.0)
    std = jnp.where(has, jnp.sqrt(jax.nn.relu(Q / d1 - sm * sm) + 1e-5),
                    SQRT1EM5)
    mn = jnp.where(has, hd + mnv, 0.0)
    mx = jnp.where(has, hd + mxv, 0.0)
    agg = jnp.concatenate([mean, mn, mx, std], axis=1)
    pa = jnp.dot(agg, wa_ref[...], preferred_element_type=jnp.float32, precision=lax.Precision.HIGHEST)
    pb = jnp.dot(agg, wb_ref[...], preferred_element_type=jnp.float32, precision=lax.Precision.HIGHEST)
    pc = jnp.dot(agg, wc_ref[...], preferred_element_type=jnp.float32, precision=lax.Precision.HIGHEST)
    z75 = (jnp.dot(h, wx_ref[...], preferred_element_type=jnp.float32, precision=lax.Precision.HIGHEST)
           + pa + amp * pb + att * pc + bp_ref[...][None, :])
    z = jnp.dot(z75, lw_ref[...], preferred_element_type=jnp.float32, precision=lax.Precision.HIGHEST) \
        + lb_ref[...][None, :]
    z_ref[...] = z

    rows = i * NB + lax.broadcasted_iota(jnp.int32, (NB, 1), 0)
    zm = jnp.where(rows < N, z, 0.0)
    acc_ref[0:1, :] += jnp.sum(zm, axis=0, keepdims=True)
    acc_ref[1:2, :] += jnp.sum(zm * zm, axis=0, keepdims=True)
    bn_ref[...] = acc_ref[...]


def _post(h, stats, cnt, wd, wx, wa, wb, wc, bp, lw, lb):
    return pl.pallas_call(
        _post_body,
        grid=(NGRID,),
        in_specs=[pl.BlockSpec((NB, 128), lambda i: (i, 0)),
                  pl.BlockSpec((NB, ROW), lambda i: (i, 0)),
                  pl.BlockSpec((NB, 128), lambda i: (i, 0)),
                  pl.BlockSpec((128, W384), lambda i: (0, 0)),
                  pl.BlockSpec((128, 128), lambda i: (0, 0)),
                  pl.BlockSpec((ROW, 128), lambda i: (0, 0)),
                  pl.BlockSpec((ROW, 128), lambda i: (0, 0)),
                  pl.BlockSpec((ROW, 128), lambda i: (0, 0)),
                  pl.BlockSpec((128,), lambda i: (0,)),
                  pl.BlockSpec((128, 128), lambda i: (0, 0)),
                  pl.BlockSpec((128,), lambda i: (0,))],
        out_specs=[pl.BlockSpec((NB, 128), lambda i: (i, 0)),
                   pl.BlockSpec((2, 128), lambda i: (0, 0))],
        out_shape=[jax.ShapeDtypeStruct((NPAD, 128), jnp.float32),
                   jax.ShapeDtypeStruct((2, 128), jnp.float32)],
        scratch_shapes=[pltpu.VMEM((2, 128), jnp.float32)],
    )(h, stats, cnt, wd, wx, wa, wb, wc, bp, lw, lb)


def _bn_body(z_ref, s_ref, g_ref, b_ref, o_ref):
    s = s_ref[...]
    mu = s[0:1, :] / float(N)
    var = s[1:2, :] / float(N) - mu * mu
    scale = g_ref[...][None, :] * lax.rsqrt(var + 1e-5)
    o_ref[...] = jax.nn.relu((z_ref[...] - mu) * scale + b_ref[...][None, :])


def _bn_relu(z, sums, g, b):
    return pl.pallas_call(
        _bn_body,
        grid=(NGRID,),
        in_specs=[pl.BlockSpec((NB, 128), lambda i: (i, 0)),
                  pl.BlockSpec((2, 128), lambda i: (0, 0)),
                  pl.BlockSpec((128,), lambda i: (0,)),
                  pl.BlockSpec((128,), lambda i: (0,))],
        out_specs=pl.BlockSpec((NB, 128), lambda i: (i, 0)),
        out_shape=jax.ShapeDtypeStruct((NPAD, 128), jnp.float32),
    )(z, sums, g, b)


def _pool_body(h_ref, b_ref, w1_ref, b1_ref, w2_ref, b2_ref, w3_ref, b3_ref,
               o_ref, acc_ref):
    i = pl.program_id(0)

    @pl.when(i == 0)
    def _():
        acc_ref[...] = jnp.zeros_like(acc_ref)

    bb = b_ref[...]
    rows = i * NB + lax.broadcasted_iota(jnp.int32, (NB, 128), 0)
    cols = lax.broadcasted_iota(jnp.int32, (NB, 128), 1)
    oh = ((bb == cols) & (rows < N)).astype(jnp.float32)
    acc_ref[...] += lax.dot_general(oh, h_ref[...],
                                    (((0,), (0,)), ((), ())),
                                    preferred_element_type=jnp.float32, precision=lax.Precision.HIGHEST)

    @pl.when(i == NGRID - 1)
    def _():
        g0 = acc_ref[...]
        g1 = jax.nn.relu(jnp.dot(g0, w1_ref[...],
                                 preferred_element_type=jnp.float32, precision=lax.Precision.HIGHEST)
                         + b1_ref[...][None, :])
        g2 = jax.nn.relu(jnp.dot(g1, w2_ref[...],
                                 preferred_element_type=jnp.float32, precision=lax.Precision.HIGHEST)
                         + b2_ref[...][None, :])
        g3 = jnp.dot(g2, w3_ref[...], preferred_element_type=jnp.float32, precision=lax.Precision.HIGHEST) \
            + b3_ref[...][None, :]
        o_ref[...] = g3[:, 0:1]


def _pool_mlp(h, batch_pad, w1, b1, w2, b2, w3, b3):
    return pl.pallas_call(
        _pool_body,
        grid=(NGRID,),
        in_specs=[pl.BlockSpec((NB, 128), lambda i: (i, 0)),
                  pl.BlockSpec((NB, 128), lambda i: (i, 0)),
                  pl.BlockSpec((128, 128), lambda i: (0, 0)),
                  pl.BlockSpec((128,), lambda i: (0,)),
                  pl.BlockSpec((128, 128), lambda i: (0, 0)),
                  pl.BlockSpec((128,), lambda i: (0,)),
                  pl.BlockSpec((128, 128), lambda i: (0, 0)),
                  pl.BlockSpec((128,), lambda i: (0,))],
        out_specs=pl.BlockSpec((G, 1), lambda i: (0, 0)),
        out_shape=jax.ShapeDtypeStruct((G, 1), jnp.float32),
        scratch_shapes=[pltpu.VMEM((128, 128), jnp.float32)],
    )(h, batch_pad, w1, b1, w2, b2, w3, b3)


CH = 32
SB = 16
SLOTS = 48
HALF = 192
NGRP = HALF // 16
FMAX = float(np.float32(3.0e38))


def _sc_walk(h, cur_in, d_e, a_e, i, accs, d0, d1, rows_v, c_v, stage_v,
             out_hbm, do_stream):
    def flush(args):
        cur = args[0]
        acc = args[1:]
        valid = jnp.logical_and(cur >= d0, cur < d1)

        @pl.when(valid)
        def _():
            slot = lax.rem(cur, SLOTS)
            for st in range(4):
                for k in range(NGRP):
                    off = slot * ROW + st * W384 + h * HALF + k * 16
                    stage_v[pl.ds(off, 16)] = acc[st * NGRP + k]
            if do_stream:
                cross = (d_e >> 4) != (cur >> 4)

                @pl.when(cross)
                def _():
                    blk = cur >> 4
                    sbase = pl.multiple_of(lax.rem(blk, 3) * (SB * ROW), 8)
                    obase = pl.multiple_of(blk * (SB * ROW), 8)
                    pltpu.sync_copy(stage_v.at[pl.ds(sbase, SB * ROW)],
                                    out_hbm.at[pl.ds(obase, SB * ROW)])

        zero = jnp.zeros((16,), jnp.float32)
        init = ([zero] * (2 * NGRP)
                + [jnp.full((16,), FMAX, jnp.float32)] * NGRP
                + [jnp.full((16,), -FMAX, jnp.float32)] * NGRP)
        return (d_e, *init)

    carry = lax.cond(d_e != cur_in, flush, lambda a: a, (cur_in, *accs))
    cur = carry[0]
    acc = list(carry[1:])
    for k in range(NGRP):
        off = h * HALF + k * 16
        row = rows_v[i, pl.ds(off, 16)]
        c = c_v[pl.ds(a_e * W384 + off, 16)]
        v = row + c
        acc[k] = acc[k] + v
        acc[NGRP + k] = acc[NGRP + k] + v * v
        acc[2 * NGRP + k] = jnp.minimum(acc[2 * NGRP + k], v)
        acc[3 * NGRP + k] = jnp.maximum(acc[3 * NGRP + k], v)
    return (cur, *acc)


def _edge_agg(hs, src_s, dst_s, attr_s, wstart, ctab_flat):
    mesh = plsc.VectorSubcoreMesh(core_axis_name="c", subcore_axis_name="s")

    @functools.partial(
        pl.kernel, mesh=mesh,
        out_type=jax.ShapeDtypeStruct((NPAD * ROW,), jnp.float32),
        scratch_types=[
            pltpu.VMEM((CH,), jnp.int32),
            pltpu.VMEM((CH + 16,), jnp.int32),
            pltpu.VMEM((CH + 16,), jnp.int32),
            pltpu.VMEM((CH, W384), jnp.float32),
            pltpu.VMEM((SLOTS * ROW,), jnp.float32),
            pltpu.VMEM((4 * W384,), jnp.float32),
            pltpu.VMEM((272,), jnp.int32),
            pltpu.SemaphoreType.DMA,
        ])
    def k(hs_hbm, src_hbm, dst_hbm, attr_hbm, ws_hbm, c_hbm, out_hbm,
          idx_v, dst_v, attr_v, rows_v, stage_v, c_v, ws_v, sem):
        wid = lax.axis_index("s") * 2 + lax.axis_index("c")
        d0 = wid * WN
        d1 = d0 + WN
        pltpu.sync_copy(ws_hbm, ws_v.at[pl.ds(0, 256)])
        pltpu.sync_copy(c_hbm, c_v)
        wwin = ws_v[pl.ds(wid * 8, 16)]
        e0 = wwin[0]
        e1 = wwin[1]
        e0a = pl.multiple_of((e0 >> 3) << 3, 8)
        nch = (e1 - e0a + CH - 1) // CH

        zero = jnp.zeros((16,), jnp.float32)
        init = ([zero] * (2 * NGRP)
                + [jnp.full((16,), FMAX, jnp.float32)] * NGRP
                + [jnp.full((16,), -FMAX, jnp.float32)] * NGRP)

        def chunk_body(ch, carry):
            cur0 = carry[0]
            cur1 = carry[1]
            acc0 = list(carry[2:2 + 4 * NGRP])
            acc1 = list(carry[2 + 4 * NGRP:])
            base = pl.multiple_of(e0a + ch * CH, 8)
            pltpu.sync_copy(src_hbm.at[pl.ds(base, CH)], idx_v)
            pltpu.sync_copy(dst_hbm.at[pl.ds(base, CH)], dst_v.at[pl.ds(0, CH)])
            pltpu.sync_copy(attr_hbm.at[pl.ds(base, CH)],
                            attr_v.at[pl.ds(0, CH)])
            pltpu.async_copy(hs_hbm.at[idx_v], rows_v, sem).wait()

            def mk_body(h, do_stream):
                def body(g, c):
                    dwin = dst_v[pl.ds(g * 8, 16)]
                    awin = attr_v[pl.ds(g * 8, 16)]
                    for j in range(8):
                        c = _sc_walk(h, c[0], dwin[j], awin[j], g * 8 + j,
                                     c[1:], d0, d1, rows_v, c_v, stage_v,
                                     out_hbm, do_stream)
                    return c
                return body

            r0 = lax.fori_loop(0, CH // 8, mk_body(0, False), (cur0, *acc0))
            r1 = lax.fori_loop(0, CH // 8, mk_body(1, True), (cur1, *acc1))
            return (r0[0], r1[0], *r0[1:], *r1[1:])

        fin = lax.fori_loop(0, nch, chunk_body,
                            (jnp.int32(-1), jnp.int32(-1), *init, *init))
        cur0 = fin[0]
        cur1 = fin[1]
        acc0 = fin[2:2 + 4 * NGRP]
        acc1 = fin[2 + 4 * NGRP:]

        valid = jnp.logical_and(cur1 >= d0, cur1 < d1)

        @pl.when(valid)
        def _():
            slot = lax.rem(cur1, SLOTS)
            for h, acc in ((0, acc0), (1, acc1)):
                for st in range(4):
                    for k2 in range(NGRP):
                        off = slot * ROW + st * W384 + h * HALF + k2 * 16
                        stage_v[pl.ds(off, 16)] = acc[st * NGRP + k2]
            blk = cur1 >> 4
            sbase = pl.multiple_of(lax.rem(blk, 3) * (SB * ROW), 8)
            obase = pl.multiple_of(blk * (SB * ROW), 8)
            pltpu.sync_copy(stage_v.at[pl.ds(sbase, SB * ROW)],
                            out_hbm.at[pl.ds(obase, SB * ROW)])

    return k(hs, src_s, dst_s, attr_s, wstart, ctab_flat)


def kernel(x, edge_index, edge_attr, batch, node_emb, edge_emb, enc_W, enc_b,
           pre_W, pre_b, post_W, post_b, lin_W, lin_b, bn_g, bn_b,
           mlp_W1, mlp_b1, mlp_W2, mlp_b2, mlp_W3, mlp_b3):
    src = edge_index[0].astype(jnp.int32)
    dst = edge_index[1].astype(jnp.int32)
    key = (dst << 16) | (edge_attr.astype(jnp.int32) << 14) | src
    key = jnp.sort(key)
    dst_s = key >> 16
    attr_s = (key >> 14) & 3
    src_s = key & 16383
    offsets = jnp.searchsorted(dst_s, jnp.arange(N + 1)).astype(jnp.int32)
    counts = (offsets[1:] - offsets[:-1]).astype(jnp.float32)
    src_sp = _pad1(src_s, EPAD)
    dst_sp = jnp.pad(dst_s, (0, EPAD - E), constant_values=NPAD)
    attr_sp = _pad1(attr_s, EPAD)
    wpairs = jnp.zeros((32, 8), jnp.int32)
    wpairs = wpairs.at[:, 0].set(offsets[jnp.arange(32) * WN])
    wpairs = wpairs.at[:, 1].set(
        offsets[jnp.minimum((jnp.arange(32) + 1) * WN, N)])
    wpairs = wpairs.reshape(-1)
    cnt_pad = jnp.broadcast_to(_pad1(counts, NPAD)[:, None], (NPAD, 128))

    x_pad = jnp.broadcast_to(
        _pad1(jnp.squeeze(x, axis=1).astype(jnp.int32), NPAD)[:, None],
        (NPAD, 128))
    batch_pad = jnp.broadcast_to(
        _pad1(batch.astype(jnp.int32), NPAD)[:, None], (NPAD, 128))
    emb_pad = _pad2(node_emb, 128, 128)

    h = _embed(x_pad, emb_pad)

    for l in range(L):
        e4 = edge_emb @ enc_W[l] + enc_b[l]
        ctab = (jnp.einsum('af,tfg->atg', e4, pre_W[l][:, 2 * F:3 * F, :])
                + pre_b[l]).reshape(4, T * F)
        ctab_pad = _pad2(ctab, 4, W384)
        ws = _pad2(pre_W[l][:, F:2 * F, :].transpose(1, 0, 2).reshape(F, T * F),
                   128, W384)
        wd = _pad2(pre_W[l][:, 0:F, :].transpose(1, 0, 2).reshape(F, T * F),
                   128, W384)
        wx = _pad2(post_W[l][:, 0:F, :].transpose(1, 0, 2).reshape(F, T * FOUT),
                   128, 128)

        def _blockdiag(wsc):
            tmp = wsc.reshape(T, 4, F, FOUT).transpose(1, 0, 2, 3)
            bd = tmp[:, :, :, None, :] * jnp.eye(T)[None, :, None, :, None]
            bd = bd.reshape(4, T * F, T * FOUT)
            return jnp.concatenate([_pad2(bd[s], W384, 128) for s in range(4)],
                                   axis=0)

        wa = _blockdiag(post_W[l][:, F + 0 * 4 * F + 0:F + 1 * 4 * F, :])
        wb = _blockdiag(post_W[l][:, F + 1 * 4 * F:F + 2 * 4 * F, :])
        wc = _blockdiag(post_W[l][:, F + 2 * 4 * F:F + 3 * 4 * F, :])
        bp = _pad1(post_b[l].reshape(T * FOUT), 128)
        lw = _pad2(lin_W[l], 128, 128)
        lb = _pad1(lin_b[l], 128)
        bg = _pad1(bn_g[l], 128)
        bb = _pad1(bn_b[l], 128)

        hs = _node_mm(h, ws)
        stats = _edge_agg(hs, src_sp, dst_sp, attr_sp, wpairs,
                          ctab_pad.reshape(-1)).reshape(NPAD, ROW)
        z, sums = _post(h, stats, cnt_pad, wd, wx, wa, wb, wc, bp, lw, lb)
        h = _bn_relu(z, sums, bg, bb)

    w1 = _pad2(mlp_W1, 128, 128)
    b1 = _pad1(mlp_b1, 128)
    w2 = _pad2(mlp_W2, 128, 128)
    b2 = _pad1(mlp_b2, 128)
    w3 = _pad2(mlp_W3, 128, 128)
    b3 = _pad1(mlp_b3, 128)
    return _pool_mlp(h, batch_pad, w1, b1, w2, b2, w3, b3)


# speedup vs baseline: 48.3769x; 1.0366x over previous
import functools
import numpy as np
import jax
import jax.numpy as jnp
from jax import lax
from jax.experimental import pallas as pl
from jax.experimental.pallas import tpu as pltpu
from jax.experimental.pallas import tpu_sc as plsc

N = 10000
E = 160000
G = 128
L = 4
T = 5
F = 75
FOUT = 15

_DEG = np.array([0, 120, 340, 780, 1450, 2300, 3200, 4100, 4800, 5200, 5300,
                 5100, 4600, 3900, 3100, 2300, 1600, 1050, 640, 360, 190, 95,
                 45, 20, 8, 2], dtype=np.float64)
AVG_LOG = float((np.log(np.arange(len(_DEG)) + 1.0) * _DEG).sum() / _DEG.sum())

NPAD = 10240
WN = 320
EPAD = E + 64
W384 = 384
ROW = 4 * W384
NB = 256
NGRID = NPAD // NB
SQRT1EM5 = float(np.sqrt(1e-5))


def _pad2(a, r, c):
    return jnp.pad(a, ((0, r - a.shape[0]), (0, c - a.shape[1])))


def _pad1(a, n):
    return jnp.pad(a, (0, n - a.shape[0]))


def _embed_body(x_ref, emb_ref, o_ref):
    xb = x_ref[...]
    cols = lax.broadcasted_iota(jnp.int32, (NB, 128), 1)
    oh = (xb == cols).astype(jnp.float32)
    o_ref[...] = jnp.dot(oh, emb_ref[...], preferred_element_type=jnp.float32, precision=lax.Precision.HIGHEST)


def _embed(x_pad, emb_pad):
    return pl.pallas_call(
        _embed_body,
        grid=(NGRID,),
        in_specs=[pl.BlockSpec((NB, 128), lambda i: (i, 0)),
                  pl.BlockSpec((128, 128), lambda i: (0, 0))],
        out_specs=pl.BlockSpec((NB, 128), lambda i: (i, 0)),
        out_shape=jax.ShapeDtypeStruct((NPAD, 128), jnp.float32),
    )(x_pad, emb_pad)


def _mm_body(h_ref, w_ref, o_ref):
    o_ref[...] = jnp.dot(h_ref[...], w_ref[...],
                         preferred_element_type=jnp.float32, precision=lax.Precision.HIGHEST)


def _node_mm(h, w):
    return pl.pallas_call(
        _mm_body,
        grid=(NGRID,),
        in_specs=[pl.BlockSpec((NB, 128), lambda i: (i, 0)),
                  pl.BlockSpec((128, W384), lambda i: (0, 0))],
        out_specs=pl.BlockSpec((NB, W384), lambda i: (i, 0)),
        out_shape=jax.ShapeDtypeStruct((NPAD, W384), jnp.float32),
    )(h, w)


def _post_body(h_ref, st_ref, cnt_ref, wd_ref, wx_ref, wa_ref, wb_ref,
               wc_ref, bp_ref, lw_ref, lb_ref, z_ref, bn_ref, acc_ref):
    i = pl.program_id(0)

    @pl.when(i == 0)
    def _():
        acc_ref[...] = jnp.zeros_like(acc_ref)

    cnt = cnt_ref[:, 0:1]
    deg = jnp.maximum(cnt, 1.0)
    logd = jnp.log(deg + 1.0)
    amp = logd / AVG_LOG
    att = AVG_LOG / logd
    has = cnt > 0.0

    h = h_ref[...]
    hd = jnp.dot(h, wd_ref[...], preferred_element_type=jnp.float32, precision=lax.Precision.HIGHEST)
    d1 = deg
    S = st_ref[:, 0:W384]
    Q = st_ref[:, W384:2 * W384]
    mnv = st_ref[:, 2 * W384:3 * W384]
    mxv = st_ref[:, 3 * W384:4 * W384]
    sm = S / d1
    mean = jnp.where(has, hd + sm, 0.0)
    std = jnp.where(has, jnp.sqrt(jax.nn.relu(Q / d1 - sm * sm) + 1e-5),
                    SQRT1EM5)
    mn = jnp.where(has, hd + mnv, 0.0)
    mx = jnp.where(has, hd + mxv, 0.0)
    agg = jnp.concatenate([mean, mn, mx, std], axis=1)
    pa = jnp.dot(agg, wa_ref[...], preferred_element_type=jnp.float32, precision=lax.Precision.HIGHEST)
    pb = jnp.dot(agg, wb_ref[...], preferred_element_type=jnp.float32, precision=lax.Precision.HIGHEST)
    pc = jnp.dot(agg, wc_ref[...], preferred_element_type=jnp.float32, precision=lax.Precision.HIGHEST)
    z75 = (jnp.dot(h, wx_ref[...], preferred_element_type=jnp.float32, precision=lax.Precision.HIGHEST)
           + pa + amp * pb + att * pc + bp_ref[...][None, :])
    z = jnp.dot(z75, lw_ref[...], preferred_element_type=jnp.float32, precision=lax.Precision.HIGHEST) \
        + lb_ref[...][None, :]
    z_ref[...] = z

    rows = i * NB + lax.broadcasted_iota(jnp.int32, (NB, 1), 0)
    zm = jnp.where(rows < N, z, 0.0)
    acc_ref[0:1, :] += jnp.sum(zm, axis=0, keepdims=True)
    acc_ref[1:2, :] += jnp.sum(zm * zm, axis=0, keepdims=True)
    bn_ref[...] = acc_ref[...]


def _post(h, stats, cnt, wd, wx, wa, wb, wc, bp, lw, lb):
    return pl.pallas_call(
        _post_body,
        grid=(NGRID,),
        in_specs=[pl.BlockSpec((NB, 128), lambda i: (i, 0)),
                  pl.BlockSpec((NB, ROW), lambda i: (i, 0)),
                  pl.BlockSpec((NB, 128), lambda i: (i, 0)),
                  pl.BlockSpec((128, W384), lambda i: (0, 0)),
                  pl.BlockSpec((128, 128), lambda i: (0, 0)),
                  pl.BlockSpec((ROW, 128), lambda i: (0, 0)),
                  pl.BlockSpec((ROW, 128), lambda i: (0, 0)),
                  pl.BlockSpec((ROW, 128), lambda i: (0, 0)),
                  pl.BlockSpec((128,), lambda i: (0,)),
                  pl.BlockSpec((128, 128), lambda i: (0, 0)),
                  pl.BlockSpec((128,), lambda i: (0,))],
        out_specs=[pl.BlockSpec((NB, 128), lambda i: (i, 0)),
                   pl.BlockSpec((2, 128), lambda i: (0, 0))],
        out_shape=[jax.ShapeDtypeStruct((NPAD, 128), jnp.float32),
                   jax.ShapeDtypeStruct((2, 128), jnp.float32)],
        scratch_shapes=[pltpu.VMEM((2, 128), jnp.float32)],
    )(h, stats, cnt, wd, wx, wa, wb, wc, bp, lw, lb)


def _bn_body(z_ref, s_ref, g_ref, b_ref, o_ref):
    s = s_ref[...]
    mu = s[0:1, :] / float(N)
    var = s[1:2, :] / float(N) - mu * mu
    o_ref[...] = jax.nn.relu((z_ref[...] - mu) / jnp.sqrt(var + 1e-5)
                             * g_ref[...][None, :] + b_ref[...][None, :])


def _bn_relu(z, sums, g, b):
    return pl.pallas_call(
        _bn_body,
        grid=(NGRID,),
        in_specs=[pl.BlockSpec((NB, 128), lambda i: (i, 0)),
                  pl.BlockSpec((2, 128), lambda i: (0, 0)),
                  pl.BlockSpec((128,), lambda i: (0,)),
                  pl.BlockSpec((128,), lambda i: (0,))],
        out_specs=pl.BlockSpec((NB, 128), lambda i: (i, 0)),
        out_shape=jax.ShapeDtypeStruct((NPAD, 128), jnp.float32),
    )(z, sums, g, b)


def _pool_body(h_ref, b_ref, w1_ref, b1_ref, w2_ref, b2_ref, w3_ref, b3_ref,
               o_ref, acc_ref):
    i = pl.program_id(0)

    @pl.when(i == 0)
    def _():
        acc_ref[...] = jnp.zeros_like(acc_ref)

    bb = b_ref[...]
    rows = i * NB + lax.broadcasted_iota(jnp.int32, (NB, 128), 0)
    cols = lax.broadcasted_iota(jnp.int32, (NB, 128), 1)
    oh = ((bb == cols) & (rows < N)).astype(jnp.float32)
    acc_ref[...] += lax.dot_general(oh, h_ref[...],
                                    (((0,), (0,)), ((), ())),
                                    preferred_element_type=jnp.float32, precision=lax.Precision.HIGHEST)

    @pl.when(i == NGRID - 1)
    def _():
        g0 = acc_ref[...]
        g1 = jax.nn.relu(jnp.dot(g0, w1_ref[...],
                                 preferred_element_type=jnp.float32, precision=lax.Precision.HIGHEST)
                         + b1_ref[...][None, :])
        g2 = jax.nn.relu(jnp.dot(g1, w2_ref[...],
                                 preferred_element_type=jnp.float32, precision=lax.Precision.HIGHEST)
                         + b2_ref[...][None, :])
        g3 = jnp.dot(g2, w3_ref[...], preferred_element_type=jnp.float32, precision=lax.Precision.HIGHEST) \
            + b3_ref[...][None, :]
        o_ref[...] = g3[:, 0:1]


def _pool_mlp(h, batch_pad, w1, b1, w2, b2, w3, b3):
    return pl.pallas_call(
        _pool_body,
        grid=(NGRID,),
        in_specs=[pl.BlockSpec((NB, 128), lambda i: (i, 0)),
                  pl.BlockSpec((NB, 128), lambda i: (i, 0)),
                  pl.BlockSpec((128, 128), lambda i: (0, 0)),
                  pl.BlockSpec((128,), lambda i: (0,)),
                  pl.BlockSpec((128, 128), lambda i: (0, 0)),
                  pl.BlockSpec((128,), lambda i: (0,)),
                  pl.BlockSpec((128, 128), lambda i: (0, 0)),
                  pl.BlockSpec((128,), lambda i: (0,))],
        out_specs=pl.BlockSpec((G, 1), lambda i: (0, 0)),
        out_shape=jax.ShapeDtypeStruct((G, 1), jnp.float32),
        scratch_shapes=[pltpu.VMEM((128, 128), jnp.float32)],
    )(h, batch_pad, w1, b1, w2, b2, w3, b3)


CH = 32
SB = 16
SLOTS = 48
HALF = 192
NGRP = HALF // 16
FMAX = float(np.float32(3.0e38))


def _sc_walk(h, cur_in, d_e, a_e, i, accs, d0, d1, rows_v, c_v, stage_v,
             out_hbm, do_stream):
    def flush(args):
        cur = args[0]
        acc = args[1:]
        valid = jnp.logical_and(cur >= d0, cur < d1)

        @pl.when(valid)
        def _():
            slot = lax.rem(cur, SLOTS)
            for st in range(4):
                for k in range(NGRP):
                    off = slot * ROW + st * W384 + h * HALF + k * 16
                    stage_v[pl.ds(off, 16)] = acc[st * NGRP + k]
            if do_stream:
                cross = (d_e >> 4) != (cur >> 4)

                @pl.when(cross)
                def _():
                    blk = cur >> 4
                    sbase = pl.multiple_of(lax.rem(blk, 3) * (SB * ROW), 8)
                    obase = pl.multiple_of(blk * (SB * ROW), 8)
                    pltpu.sync_copy(stage_v.at[pl.ds(sbase, SB * ROW)],
                                    out_hbm.at[pl.ds(obase, SB * ROW)])

        zero = jnp.zeros((16,), jnp.float32)
        init = ([zero] * (2 * NGRP)
                + [jnp.full((16,), FMAX, jnp.float32)] * NGRP
                + [jnp.full((16,), -FMAX, jnp.float32)] * NGRP)
        return (d_e, *init)

    carry = lax.cond(d_e != cur_in, flush, lambda a: a, (cur_in, *accs))
    cur = carry[0]
    acc = list(carry[1:])
    for k in range(NGRP):
        off = h * HALF + k * 16
        row = rows_v[i, pl.ds(off, 16)]
        c = c_v[pl.ds(a_e * W384 + off, 16)]
        v = row + c
        acc[k] = acc[k] + v
        acc[NGRP + k] = acc[NGRP + k] + v * v
        acc[2 * NGRP + k] = jnp.minimum(acc[2 * NGRP + k], v)
        acc[3 * NGRP + k] = jnp.maximum(acc[3 * NGRP + k], v)
    return (cur, *acc)


def _edge_agg(hs, src_s, comb_s, wstart, ctab_flat):
    mesh = plsc.VectorSubcoreMesh(core_axis_name="c", subcore_axis_name="s")

    @functools.partial(
        pl.kernel, mesh=mesh,
        out_type=jax.ShapeDtypeStruct((NPAD * ROW,), jnp.float32),
        scratch_types=[
            pltpu.VMEM((CH,), jnp.int32),
            pltpu.VMEM((CH + 16,), jnp.int32),
            pltpu.VMEM((CH, W384), jnp.float32),
            pltpu.VMEM((SLOTS * ROW,), jnp.float32),
            pltpu.VMEM((4 * W384,), jnp.float32),
            pltpu.VMEM((272,), jnp.int32),
            pltpu.SemaphoreType.DMA,
        ])
    def k(hs_hbm, src_hbm, comb_hbm, ws_hbm, c_hbm, out_hbm,
          idx_v, comb_v, rows_v, stage_v, c_v, ws_v, sem):
        wid = lax.axis_index("s") * 2 + lax.axis_index("c")
        d0 = wid * WN
        d1 = d0 + WN
        pltpu.sync_copy(ws_hbm, ws_v.at[pl.ds(0, 256)])
        pltpu.sync_copy(c_hbm, c_v)
        wwin = ws_v[pl.ds(wid * 8, 16)]
        e0 = wwin[0]
        e1 = wwin[1]
        e0a = pl.multiple_of((e0 >> 3) << 3, 8)
        nch = (e1 - e0a + CH - 1) // CH

        zero = jnp.zeros((16,), jnp.float32)
        init = ([zero] * (2 * NGRP)
                + [jnp.full((16,), FMAX, jnp.float32)] * NGRP
                + [jnp.full((16,), -FMAX, jnp.float32)] * NGRP)

        def chunk_body(ch, carry):
            cur0 = carry[0]
            cur1 = carry[1]
            acc0 = list(carry[2:2 + 4 * NGRP])
            acc1 = list(carry[2 + 4 * NGRP:])
            base = pl.multiple_of(e0a + ch * CH, 8)
            pltpu.sync_copy(src_hbm.at[pl.ds(base, CH)], idx_v)
            pltpu.sync_copy(comb_hbm.at[pl.ds(base, CH)],
                            comb_v.at[pl.ds(0, CH)])
            pltpu.async_copy(hs_hbm.at[idx_v], rows_v, sem).wait()

            def mk_body(h, do_stream):
                def body(g, c):
                    mwin = comb_v[pl.ds(g * 8, 16)]
                    for j in range(8):
                        c = _sc_walk(h, c[0], mwin[j] >> 2, mwin[j] & 3,
                                     g * 8 + j, c[1:], d0, d1, rows_v, c_v,
                                     stage_v, out_hbm, do_stream)
                    return c
                return body

            r0 = lax.fori_loop(0, CH // 8, mk_body(0, False), (cur0, *acc0))
            r1 = lax.fori_loop(0, CH // 8, mk_body(1, True), (cur1, *acc1))
            return (r0[0], r1[0], *r0[1:], *r1[1:])

        fin = lax.fori_loop(0, nch, chunk_body,
                            (jnp.int32(-1), jnp.int32(-1), *init, *init))
        cur0 = fin[0]
        cur1 = fin[1]
        acc0 = fin[2:2 + 4 * NGRP]
        acc1 = fin[2 + 4 * NGRP:]

        valid = jnp.logical_and(cur1 >= d0, cur1 < d1)

        @pl.when(valid)
        def _():
            slot = lax.rem(cur1, SLOTS)
            for h, acc in ((0, acc0), (1, acc1)):
                for st in range(4):
                    for k2 in range(NGRP):
                        off = slot * ROW + st * W384 + h * HALF + k2 * 16
                        stage_v[pl.ds(off, 16)] = acc[st * NGRP + k2]
            blk = cur1 >> 4
            sbase = pl.multiple_of(lax.rem(blk, 3) * (SB * ROW), 8)
            obase = pl.multiple_of(blk * (SB * ROW), 8)
            pltpu.sync_copy(stage_v.at[pl.ds(sbase, SB * ROW)],
                            out_hbm.at[pl.ds(obase, SB * ROW)])

    return k(hs, src_s, comb_s, wstart, ctab_flat)


def kernel(x, edge_index, edge_attr, batch, node_emb, edge_emb, enc_W, enc_b,
           pre_W, pre_b, post_W, post_b, lin_W, lin_b, bn_g, bn_b,
           mlp_W1, mlp_b1, mlp_W2, mlp_b2, mlp_W3, mlp_b3):
    src = edge_index[0].astype(jnp.int32)
    dst = edge_index[1].astype(jnp.int32)
    key = (dst << 16) | (edge_attr.astype(jnp.int32) << 14) | src
    key = jnp.sort(key)
    dst_s = key >> 16
    attr_s = (key >> 14) & 3
    src_s = key & 16383
    offsets = jnp.searchsorted(dst_s, jnp.arange(N + 1)).astype(jnp.int32)
    counts = (offsets[1:] - offsets[:-1]).astype(jnp.float32)
    src_sp = _pad1(src_s, EPAD)
    comb_sp = jnp.pad(key >> 14, (0, EPAD - E), constant_values=NPAD * 4)
    wpairs = jnp.zeros((32, 8), jnp.int32)
    wpairs = wpairs.at[:, 0].set(offsets[jnp.arange(32) * WN])
    wpairs = wpairs.at[:, 1].set(
        offsets[jnp.minimum((jnp.arange(32) + 1) * WN, N)])
    wpairs = wpairs.reshape(-1)
    cnt_pad = jnp.broadcast_to(_pad1(counts, NPAD)[:, None], (NPAD, 128))

    x_pad = jnp.broadcast_to(
        _pad1(jnp.squeeze(x, axis=1).astype(jnp.int32), NPAD)[:, None],
        (NPAD, 128))
    batch_pad = jnp.broadcast_to(
        _pad1(batch.astype(jnp.int32), NPAD)[:, None], (NPAD, 128))
    emb_pad = _pad2(node_emb, 128, 128)

    h = _embed(x_pad, emb_pad)

    for l in range(L):
        e4 = jnp.dot(edge_emb, enc_W[l],
                     precision=lax.Precision.HIGHEST) + enc_b[l]
        ctab = (jnp.einsum('af,tfg->atg', e4, pre_W[l][:, 2 * F:3 * F, :],
                           precision=lax.Precision.HIGHEST)
                + pre_b[l]).reshape(4, T * F)
        ctab_pad = _pad2(ctab, 4, W384)
        ws = _pad2(pre_W[l][:, F:2 * F, :].transpose(1, 0, 2).reshape(F, T * F),
                   128, W384)
        wd = _pad2(pre_W[l][:, 0:F, :].transpose(1, 0, 2).reshape(F, T * F),
                   128, W384)
        wx = _pad2(post_W[l][:, 0:F, :].transpose(1, 0, 2).reshape(F, T * FOUT),
                   128, 128)

        def _blockdiag(wsc):
            tmp = wsc.reshape(T, 4, F, FOUT).transpose(1, 0, 2, 3)
            bd = tmp[:, :, :, None, :] * jnp.eye(T)[None, :, None, :, None]
            bd = bd.reshape(4, T * F, T * FOUT)
            return jnp.concatenate([_pad2(bd[s], W384, 128) for s in range(4)],
                                   axis=0)

        wa = _blockdiag(post_W[l][:, F + 0 * 4 * F + 0:F + 1 * 4 * F, :])
        wb = _blockdiag(post_W[l][:, F + 1 * 4 * F:F + 2 * 4 * F, :])
        wc = _blockdiag(post_W[l][:, F + 2 * 4 * F:F + 3 * 4 * F, :])
        bp = _pad1(post_b[l].reshape(T * FOUT), 128)
        lw = _pad2(lin_W[l], 128, 128)
        lb = _pad1(lin_b[l], 128)
        bg = _pad1(bn_g[l], 128)
        bb = _pad1(bn_b[l], 128)

        hs = _node_mm(h, ws)
        stats = _edge_agg(hs, src_sp, comb_sp, wpairs,
                          ctab_pad.reshape(-1)).reshape(NPAD, ROW)
        z, sums = _post(h, stats, cnt_pad, wd, wx, wa, wb, wc, bp, lw, lb)
        h = _bn_relu(z, sums, bg, bb)

    w1 = _pad2(mlp_W1, 128, 128)
    b1 = _pad1(mlp_b1, 128)
    w2 = _pad2(mlp_W2, 128, 128)
    b2 = _pad1(mlp_b2, 128)
    w3 = _pad2(mlp_W3, 128, 128)
    b3 = _pad1(mlp_b3, 128)
    return _pool_mlp(h, batch_pad, w1, b1, w2, b2, w3, b3)
